# baseline (speedup 1.0000x reference)
import functools

import jax
import jax.numpy as jnp
from jax import lax
from jax.experimental import pallas as pl
from jax.experimental.pallas import tpu as pltpu
from jax.experimental.pallas import tpu_sc as plsc

N = 10000
E = 320000
D = 128
NC, NS = 2, 16
NW = NC * NS
EPW = E // NW
C = 112
NCH = EPW // C
CT = EPW - NCH * C
CD = 2000
NP = 10112
P = NP // NS
BN = 1000
BLOCKS = N // BN
SCALE = 0.125


@functools.cache
def _build_sc_aggregate():
    mesh = plsc.VectorSubcoreMesh(core_axis_name="c", subcore_axis_name="s",
                                  num_cores=NC)
    return pl.kernel(
        _sc_aggregate_body,
        out_type=jax.ShapeDtypeStruct((NC, NP, D), jnp.float32),
        mesh=mesh,
        scratch_types=[
            pltpu.VMEM((EPW,), jnp.int32),
            pltpu.VMEM((EPW,), jnp.int32),
            pltpu.VMEM((C,), jnp.int32),
            pltpu.VMEM((C,), jnp.int32),
            pltpu.VMEM((C,), jnp.int32),
            pltpu.VMEM((C,), jnp.int32),
            pltpu.VMEM((CT,), jnp.int32),
            pltpu.VMEM((CT,), jnp.int32),
            pltpu.VMEM((C, D), jnp.float32),
            pltpu.VMEM((C, D), jnp.float32),
            pltpu.VMEM_SHARED((NP, D), jnp.float32),
            pltpu.SemaphoreType.DMA,
            pltpu.SemaphoreType.DMA,
            pltpu.SemaphoreType.DMA,
            pltpu.SemaphoreType.DMA,
        ],
    )


def _sc_aggregate_body(y, srcs, dsts, out,
                       src_all, dst_all, src_v0, src_v1, dst_v0, dst_v1,
                       src_vt, dst_vt, rows_v0, rows_v1, acc,
                       sg0, sg1, ss0, ss1):
    c = lax.axis_index("c")
    s = lax.axis_index("s")
    w = s * NC + c
    src_v = (src_v0, src_v1)
    dst_v = (dst_v0, dst_v1)
    rows_v = (rows_v0, rows_v1)
    sem_g = (sg0, sg1)
    sem_s = (ss0, ss1)

    pltpu.async_copy(srcs.at[pl.ds(w * EPW, EPW)], src_all, sg0)
    pltpu.async_copy(dsts.at[pl.ds(w * EPW, EPW)], dst_all, sg1)

    zero16 = jnp.zeros((16,), jnp.float32)

    def _zrow(i, carry):
        def _zcol(j, cc):
            rows_v1[i, pl.ds(j * 16, 16)] = zero16
            return cc
        return lax.fori_loop(0, D // 16, _zcol, carry)

    lax.fori_loop(0, C, _zrow, 0)
    for j in range(P // C):
        pltpu.async_copy(rows_v1, acc.at[pl.ds(s * P + j * C, C)], ss1)
    rem = P % C
    if rem:
        pltpu.async_copy(rows_v1.at[pl.ds(0, rem)],
                         acc.at[pl.ds(s * P + (P // C) * C, rem)], ss1)
    pltpu.make_async_copy(srcs.at[pl.ds(w * EPW, EPW)], src_all, sg0).wait()
    pltpu.make_async_copy(dsts.at[pl.ds(w * EPW, EPW)], dst_all, sg1).wait()

    def _start_gather(k, b):
        for g in range(C // 16):
            src_v[b][pl.ds(g * 16, 16)] = src_all[pl.ds(k * C + g * 16, 16)]
            dst_v[b][pl.ds(g * 16, 16)] = dst_all[pl.ds(k * C + g * 16, 16)]
        h = C // 2
        pltpu.async_copy(y.at[src_v[b].at[pl.ds(0, h)]],
                         rows_v[b].at[pl.ds(0, h)], sem_g[b])
        pltpu.async_copy(y.at[src_v[b].at[pl.ds(h, h)]],
                         rows_v[b].at[pl.ds(h, h)], sem_g[b])

    _start_gather(0, 0)
    for j in range(P // C):
        pltpu.make_async_copy(rows_v1, acc.at[pl.ds(s * P + j * C, C)],
                              ss1).wait()
    if rem:
        pltpu.make_async_copy(rows_v1.at[pl.ds(0, rem)],
                              acc.at[pl.ds(s * P + (P // C) * C, rem)],
                              ss1).wait()
    plsc.subcore_barrier()
    _start_gather(1, 1)

    def _step(k2, carry):
        for b in range(2):
            k = k2 * 2 + b
            pltpu.make_async_copy(y.at[src_v[b]], rows_v[b], sem_g[b]).wait()
            pltpu.async_copy(rows_v[b], acc.at[dst_v[b]], sem_s[b],
                             add=True).wait()

            @pl.when(k + 2 < NCH)
            def _():
                _start_gather(k + 2, b)
        return carry

    lax.fori_loop(0, NCH // 2, _step, 0)

    if NCH % 2:
        kl = NCH - 1
        bl = kl % 2
        pltpu.make_async_copy(y.at[src_v[bl]], rows_v[bl], sem_g[bl]).wait()
        pltpu.async_copy(rows_v[bl], acc.at[dst_v[bl]], sem_s[bl],
                         add=True).wait()

    if CT:
        base = NCH * C
        for g in range(CT // 16):
            src_vt[pl.ds(g * 16, 16)] = src_all[pl.ds(base + g * 16, 16)]
            dst_vt[pl.ds(g * 16, 16)] = dst_all[pl.ds(base + g * 16, 16)]
        pltpu.async_copy(y.at[src_vt], rows_v0.at[pl.ds(0, CT)], sem_g[0])
        pltpu.make_async_copy(y.at[src_vt], rows_v0.at[pl.ds(0, CT)],
                              sem_g[0]).wait()
        pltpu.async_copy(rows_v0.at[pl.ds(0, CT)], acc.at[dst_vt], sem_s[0],
                         add=True).wait()

    plsc.subcore_barrier()
    pltpu.sync_copy(acc.at[pl.ds(s * P, P)], out.at[c, pl.ds(s * P, P)])


@functools.cache
def _build_sc_degree():
    mesh = plsc.VectorSubcoreMesh(core_axis_name="c", subcore_axis_name="s",
                                  num_cores=NC)
    return pl.kernel(
        _sc_degree_body,
        out_type=jax.ShapeDtypeStruct((NW * NP,), jnp.float32),
        mesh=mesh,
        scratch_types=[
            pltpu.VMEM((EPW,), jnp.int32),
            pltpu.VMEM((NP,), jnp.float32),
        ],
        compiler_params=pltpu.CompilerParams(needs_layout_passes=False),
    )


def _sc_degree_body(dsts, out, dst_v, hist):
    c = lax.axis_index("c")
    s = lax.axis_index("s")
    w = s * NC + c

    zero16 = jnp.zeros((16,), jnp.float32)
    ones16 = jnp.ones((16,), jnp.float32)

    pltpu.sync_copy(dsts.at[pl.ds(w * EPW, EPW)], dst_v)

    def _z(i, carry):
        hist[pl.ds(i * 16, 16)] = zero16
        return carry

    lax.fori_loop(0, NP // 16, _z, 0)

    def _group(g, cc):
        idx = dst_v[pl.ds(g * 16, 16)]
        plsc.addupdate_scatter(hist, [idx], ones16)
        return cc

    lax.fori_loop(0, EPW // 16, _group, 0)
    pltpu.sync_copy(hist, out.at[pl.ds(w * NP, NP)])


def _tc_layer_body(acc_ref, degt_ref, w_ref, out_ref):
    a = acc_ref[0] + acc_ref[1]
    deg = jnp.sum(degt_ref[...], axis=1, keepdims=True)
    feat = a / jnp.maximum(deg, 1.0)
    out_ref[...] = jnp.maximum(
        jnp.dot(feat, w_ref[...], preferred_element_type=jnp.float32), 0.0)


def _tc_layer(acc, degt, W):
    return pl.pallas_call(
        _tc_layer_body,
        grid=(BLOCKS,),
        in_specs=[pl.BlockSpec((NC, BN, D), lambda i: (0, i, 0)),
                  pl.BlockSpec((BN, NW), lambda i: (i, 0)),
                  pl.BlockSpec((D, D), lambda i: (0, 0))],
        out_specs=pl.BlockSpec((BN, D), lambda i: (i, 0)),
        out_shape=jax.ShapeDtypeStruct((N, D), jnp.float32),
    )(acc, degt, W)


def _tc_final_body(acc_ref, degt_ref, w_ref, out_ref, g_v, st_ref):
    p = pl.program_id(0)
    i = pl.program_id(1)

    @pl.when((p == 0) & (i == 0))
    def _():
        st_ref[...] = jnp.zeros_like(st_ref)

    @pl.when(p == 0)
    def _():
        a = acc_ref[0] + acc_ref[1]
        deg = jnp.sum(degt_ref[...], axis=1, keepdims=True)
        feat = a / jnp.maximum(deg, 1.0)
        g = jnp.maximum(
            jnp.dot(feat, w_ref[...], preferred_element_type=jnp.float32),
            0.0)
        g_v[pl.ds(i * BN, BN), :] = g
        st_ref[...] += jnp.concatenate(
            [jnp.sum(g, axis=0, keepdims=True),
             jnp.sum(g * g, axis=0, keepdims=True),
             jnp.zeros((6, D), jnp.float32)], axis=0)

    @pl.when(p == 1)
    def _():
        st = st_ref[...]
        mean = st[0:1, :] / N
        var = (st[1:2, :] - mean * mean * N) / (N - 1)
        inv = SCALE / (jnp.sqrt(jnp.maximum(var, 0.0)) + 1e-6)
        out_ref[...] = (g_v[pl.ds(i * BN, BN), :] - mean) * inv


def _tc_final(acc, degt, W):
    return pl.pallas_call(
        _tc_final_body,
        grid=(2, BLOCKS),
        in_specs=[pl.BlockSpec((NC, BN, D), lambda p, i: (0, i, 0)),
                  pl.BlockSpec((BN, NW), lambda p, i: (i, 0)),
                  pl.BlockSpec((D, D), lambda p, i: (0, 0))],
        out_specs=pl.BlockSpec((BN, D), lambda p, i: (i, 0)),
        out_shape=jax.ShapeDtypeStruct((N, D), jnp.float32),
        scratch_shapes=[pltpu.VMEM((N, D), jnp.float32),
                        pltpu.VMEM((8, D), jnp.float32)],
    )(acc, degt, W)


def kernel(x, edge_index, W1, W2):
    src = edge_index[0]
    dst = edge_index[1]
    degt = _build_sc_degree()(dst).reshape(NW, NP).T
    sc_aggregate = _build_sc_aggregate()
    acc1 = sc_aggregate(x.astype(jnp.float32), src, dst)
    h = _tc_layer(acc1, degt, W1)
    acc2 = sc_aggregate(h, src, dst)
    return _tc_final(acc2, degt, W2)


# speedup vs baseline: 1.0016x; 1.0016x over previous
import functools

import jax
import jax.numpy as jnp
from jax import lax
from jax.experimental import pallas as pl
from jax.experimental.pallas import tpu as pltpu
from jax.experimental.pallas import tpu_sc as plsc

N = 10000
E = 320000
D = 128
NC, NS = 2, 16
NW = NC * NS
EPW = E // NW
C = 112
NCH = EPW // C
CT = EPW - NCH * C
NP = 10112
P = NP // NS
BN = 1000
BLOCKS = N // BN
SCALE = 0.125


@functools.cache
def _build_sc_aggregate():
    mesh = plsc.VectorSubcoreMesh(core_axis_name="c", subcore_axis_name="s",
                                  num_cores=NC)
    return pl.kernel(
        _sc_aggregate_body,
        out_type=jax.ShapeDtypeStruct((NC, NP, D), jnp.float32),
        mesh=mesh,
        scratch_types=[
            pltpu.VMEM((EPW,), jnp.int32),
            pltpu.VMEM((EPW,), jnp.int32),
            pltpu.VMEM((C,), jnp.int32),
            pltpu.VMEM((C,), jnp.int32),
            pltpu.VMEM((C,), jnp.int32),
            pltpu.VMEM((C,), jnp.int32),
            pltpu.VMEM((CT,), jnp.int32),
            pltpu.VMEM((CT,), jnp.int32),
            pltpu.VMEM((C, D), jnp.float32),
            pltpu.VMEM((C, D), jnp.float32),
            pltpu.VMEM_SHARED((NP, D), jnp.float32),
            pltpu.SemaphoreType.DMA,
            pltpu.SemaphoreType.DMA,
            pltpu.SemaphoreType.DMA,
            pltpu.SemaphoreType.DMA,
        ],
    )


def _sc_aggregate_body(y, srcs, dsts, out,
                       src_all, dst_all, src_v0, src_v1, dst_v0, dst_v1,
                       src_vt, dst_vt, rows_v0, rows_v1, acc,
                       sg0, sg1, ss0, ss1):
    c = lax.axis_index("c")
    s = lax.axis_index("s")
    w = s * NC + c
    src_v = (src_v0, src_v1)
    dst_v = (dst_v0, dst_v1)
    rows_v = (rows_v0, rows_v1)
    sem_g = (sg0, sg1)
    sem_s = (ss0, ss1)

    pltpu.async_copy(srcs.at[pl.ds(w * EPW, EPW)], src_all, sg0)
    pltpu.async_copy(dsts.at[pl.ds(w * EPW, EPW)], dst_all, sg1)

    zero16 = jnp.zeros((16,), jnp.float32)

    def _zrow(i, carry):
        def _zcol(j, cc):
            rows_v1[i, pl.ds(j * 16, 16)] = zero16
            return cc
        return lax.fori_loop(0, D // 16, _zcol, carry)

    lax.fori_loop(0, C, _zrow, 0)
    for j in range(P // C):
        pltpu.async_copy(rows_v1, acc.at[pl.ds(s * P + j * C, C)], ss1)
    rem = P % C
    if rem:
        pltpu.async_copy(rows_v1.at[pl.ds(0, rem)],
                         acc.at[pl.ds(s * P + (P // C) * C, rem)], ss1)
    pltpu.make_async_copy(srcs.at[pl.ds(w * EPW, EPW)], src_all, sg0).wait()
    pltpu.make_async_copy(dsts.at[pl.ds(w * EPW, EPW)], dst_all, sg1).wait()

    def _start_gather(k, b):
        for g in range(C // 16):
            src_v[b][pl.ds(g * 16, 16)] = src_all[pl.ds(k * C + g * 16, 16)]
            dst_v[b][pl.ds(g * 16, 16)] = dst_all[pl.ds(k * C + g * 16, 16)]
        pltpu.async_copy(y.at[src_v[b]], rows_v[b], sem_g[b])

    _start_gather(0, 0)
    for j in range(P // C):
        pltpu.make_async_copy(rows_v1, acc.at[pl.ds(s * P + j * C, C)],
                              ss1).wait()
    if rem:
        pltpu.make_async_copy(rows_v1.at[pl.ds(0, rem)],
                              acc.at[pl.ds(s * P + (P // C) * C, rem)],
                              ss1).wait()
    plsc.subcore_barrier()
    _start_gather(1, 1)

    def _step(k2, carry):
        for b in range(2):
            k = k2 * 2 + b
            pltpu.make_async_copy(y.at[src_v[b]], rows_v[b], sem_g[b]).wait()
            pltpu.async_copy(rows_v[b], acc.at[dst_v[b]], sem_s[b],
                             add=True).wait()

            @pl.when(k + 2 < NCH)
            def _():
                _start_gather(k + 2, b)
        return carry

    lax.fori_loop(0, NCH // 2, _step, 0)

    if NCH % 2:
        kl = NCH - 1
        bl = kl % 2
        pltpu.make_async_copy(y.at[src_v[bl]], rows_v[bl], sem_g[bl]).wait()
        pltpu.async_copy(rows_v[bl], acc.at[dst_v[bl]], sem_s[bl],
                         add=True).wait()

    if CT:
        base = NCH * C
        for g in range(CT // 16):
            src_vt[pl.ds(g * 16, 16)] = src_all[pl.ds(base + g * 16, 16)]
            dst_vt[pl.ds(g * 16, 16)] = dst_all[pl.ds(base + g * 16, 16)]
        pltpu.async_copy(y.at[src_vt], rows_v0.at[pl.ds(0, CT)], sem_g[0])
        pltpu.make_async_copy(y.at[src_vt], rows_v0.at[pl.ds(0, CT)],
                              sem_g[0]).wait()
        pltpu.async_copy(rows_v0.at[pl.ds(0, CT)], acc.at[dst_vt], sem_s[0],
                         add=True).wait()

    plsc.subcore_barrier()
    pltpu.sync_copy(acc.at[pl.ds(s * P, P)], out.at[c, pl.ds(s * P, P)])


@functools.cache
def _build_sc_degree():
    mesh = plsc.VectorSubcoreMesh(core_axis_name="c", subcore_axis_name="s",
                                  num_cores=NC)
    return pl.kernel(
        _sc_degree_body,
        out_type=jax.ShapeDtypeStruct((NW * NP,), jnp.float32),
        mesh=mesh,
        scratch_types=[
            pltpu.VMEM((EPW,), jnp.int32),
            pltpu.VMEM((NP,), jnp.float32),
        ],
        compiler_params=pltpu.CompilerParams(needs_layout_passes=False),
    )


def _sc_degree_body(dsts, out, dst_v, hist):
    c = lax.axis_index("c")
    s = lax.axis_index("s")
    w = s * NC + c

    zero16 = jnp.zeros((16,), jnp.float32)
    ones16 = jnp.ones((16,), jnp.float32)

    pltpu.sync_copy(dsts.at[pl.ds(w * EPW, EPW)], dst_v)

    def _z(i, carry):
        hist[pl.ds(i * 16, 16)] = zero16
        return carry

    lax.fori_loop(0, NP // 16, _z, 0)

    def _group(g, cc):
        idx = dst_v[pl.ds(g * 16, 16)]
        plsc.addupdate_scatter(hist, [idx], ones16)
        return cc

    lax.fori_loop(0, EPW // 16, _group, 0)
    pltpu.sync_copy(hist, out.at[pl.ds(w * NP, NP)])


def _tc_layer_body(acc_ref, degt_ref, w_ref, out_ref):
    a = acc_ref[0] + acc_ref[1]
    deg = jnp.sum(degt_ref[...], axis=1, keepdims=True)
    feat = a / jnp.maximum(deg, 1.0)
    out_ref[...] = jnp.maximum(
        jnp.dot(feat, w_ref[...], preferred_element_type=jnp.float32), 0.0)


def _tc_layer(acc, degt, W):
    return pl.pallas_call(
        _tc_layer_body,
        grid=(BLOCKS,),
        in_specs=[pl.BlockSpec((NC, BN, D), lambda i: (0, i, 0)),
                  pl.BlockSpec((BN, NW), lambda i: (i, 0)),
                  pl.BlockSpec((D, D), lambda i: (0, 0))],
        out_specs=pl.BlockSpec((BN, D), lambda i: (i, 0)),
        out_shape=jax.ShapeDtypeStruct((N, D), jnp.float32),
    )(acc, degt, W)


def _tc_final_body(acc_ref, degt_ref, w_ref, out_ref, g_v, st_ref):
    p = pl.program_id(0)
    i = pl.program_id(1)

    @pl.when((p == 0) & (i == 0))
    def _():
        st_ref[...] = jnp.zeros_like(st_ref)

    @pl.when(p == 0)
    def _():
        a = acc_ref[0] + acc_ref[1]
        deg = jnp.sum(degt_ref[...], axis=1, keepdims=True)
        feat = a / jnp.maximum(deg, 1.0)
        g = jnp.maximum(
            jnp.dot(feat, w_ref[...], preferred_element_type=jnp.float32),
            0.0)
        g_v[pl.ds(i * BN, BN), :] = g
        st_ref[...] += jnp.concatenate(
            [jnp.sum(g, axis=0, keepdims=True),
             jnp.sum(g * g, axis=0, keepdims=True),
             jnp.zeros((6, D), jnp.float32)], axis=0)

    @pl.when(p == 1)
    def _():
        st = st_ref[...]
        mean = st[0:1, :] / N
        var = (st[1:2, :] - mean * mean * N) / (N - 1)
        inv = SCALE / (jnp.sqrt(jnp.maximum(var, 0.0)) + 1e-6)
        out_ref[...] = (g_v[pl.ds(i * BN, BN), :] - mean) * inv


def _tc_final(acc, degt, W):
    return pl.pallas_call(
        _tc_final_body,
        grid=(2, BLOCKS),
        in_specs=[pl.BlockSpec((NC, BN, D), lambda p, i: (0, i, 0)),
                  pl.BlockSpec((BN, NW), lambda p, i: (i, 0)),
                  pl.BlockSpec((D, D), lambda p, i: (0, 0))],
        out_specs=pl.BlockSpec((BN, D), lambda p, i: (i, 0)),
        out_shape=jax.ShapeDtypeStruct((N, D), jnp.float32),
        scratch_shapes=[pltpu.VMEM((N, D), jnp.float32),
                        pltpu.VMEM((8, D), jnp.float32)],
    )(acc, degt, W)


def kernel(x, edge_index, W1, W2):
    src = edge_index[0]
    dst = edge_index[1]
    degt = _build_sc_degree()(dst).reshape(NW, NP).T
    sc_aggregate = _build_sc_aggregate()
    acc1 = sc_aggregate(x.astype(jnp.float32), src, dst)
    h = _tc_layer(acc1, degt, W1)
    acc2 = sc_aggregate(h, src, dst)
    return _tc_final(acc2, degt, W2)
